# Initial kernel scaffold; baseline (speedup 1.0000x reference)
import jax
import jax.numpy as jnp
from jax.experimental import pallas as pl


def kernel(inputs, table):
    raise NotImplementedError("write your pallas kernel here")


# speedup vs baseline: 1.6917x; 1.6917x over previous
import functools

import jax
import jax.numpy as jnp
from jax import lax
from jax.experimental import pallas as pl
from jax.experimental.pallas import tpu as pltpu
from jax.experimental.pallas import tpu_sc as plsc

VOCAB = 1000000
EMBED_DIM = 32
BATCH = 16384
CTX = 20

NW = 32
ROWS_PER_W = BATCH // NW
CHUNK = 64
NCHUNK = ROWS_PER_W // CHUNK
IDX_PER_CHUNK = CHUNK * CTX
GATHERS = IDX_PER_CHUNK // 128


def _sc_cbow(idx_hbm, table_hbm, out_hbm, idx_v, rows_v, out_v, sem):
    nc = 2
    wid = lax.axis_index("s") * nc + lax.axis_index("c")
    base = wid * ROWS_PER_W
    inv_ctx = jnp.float32(1.0 / CTX)

    def chunk_body(c, _):
        pltpu.sync_copy(idx_hbm.at[wid, c], idx_v)
        copies = [
            pltpu.async_copy(
                table_hbm.at[idx_v.at[k]],
                rows_v.at[pl.ds(k * 128, 128)],
                sem,
            )
            for k in range(GATHERS)
        ]
        for cp in copies:
            cp.wait()

        def item_body(i, _):
            r0 = i * CTX
            acc0 = rows_v[r0, pl.ds(0, 16)]
            acc1 = rows_v[r0, pl.ds(16, 16)]
            for j in range(1, CTX):
                acc0 = acc0 + rows_v[r0 + j, pl.ds(0, 16)]
                acc1 = acc1 + rows_v[r0 + j, pl.ds(16, 16)]
            out_v[i, pl.ds(0, 16)] = acc0 * inv_ctx
            out_v[i, pl.ds(16, 16)] = acc1 * inv_ctx
            return 0

        lax.fori_loop(0, CHUNK, item_body, 0)
        pltpu.sync_copy(out_v, out_hbm.at[pl.ds(base + c * CHUNK, CHUNK)])
        return 0

    lax.fori_loop(0, NCHUNK, chunk_body, 0)


@functools.lru_cache(maxsize=1)
def _build_call():
    return functools.partial(
        pl.kernel,
        mesh=plsc.VectorSubcoreMesh(core_axis_name="c", subcore_axis_name="s"),
        out_type=jax.ShapeDtypeStruct((BATCH, EMBED_DIM), jnp.float32),
        scratch_types=[
            pltpu.VMEM((GATHERS, 128), jnp.int32),
            pltpu.VMEM((IDX_PER_CHUNK, EMBED_DIM), jnp.float32),
            pltpu.VMEM((CHUNK, EMBED_DIM), jnp.float32),
            pltpu.SemaphoreType.DMA,
        ],
        compiler_params=pltpu.CompilerParams(use_tc_tiling_on_sc=False),
    )(_sc_cbow)


def kernel(inputs, table):
    idx = inputs.astype(jnp.int32).reshape(NW, NCHUNK, GATHERS, 128)
    return _build_call()(idx, table)
